# baseline (speedup 1.0000x reference)
import functools

import jax
import jax.numpy as jnp
from jax import lax
from jax.experimental import pallas as pl
from jax.experimental.pallas import tpu as pltpu
from jax.experimental.pallas import tpu_sc as plsc

N = 4096
D = 128
TOPK = 5
BLK = 256
NSPLIT = 2
NROWS = N // NSPLIT
NB = NROWS // BLK
BIG = 3.0e38

NC = 2
NS = 16
NW = NC * NS
ROWS_PER = NROWS // NW
RI = 8
NG = ROWS_PER // RI
LANES = 16
HALF = N // 2


def _dist_body(h, erow_ref, efull_ref, out_ref, sq_ref):
    i = pl.program_id(0)

    @pl.when(i == 0)
    def _():
        ef = efull_ref[...]
        sq_ref[...] = jnp.sum(ef * ef, axis=1)[None, :]

    er = erow_ref[...]
    sq_rows = jnp.sum(er * er, axis=1)
    dot = jax.lax.dot_general(
        er, efull_ref[...], (((1,), (1,)), ((), ())),
        preferred_element_type=jnp.float32)
    out_ref[...] = (sq_rows[:, None] - 2.0 * dot) + sq_ref[...]


def _sc_body(h, dist_hbm, lab_hbm, out_hbm, rows_v, lab_v, acc_v, sem0, sem1):
    c = lax.axis_index("c")
    s = lax.axis_index("s")
    wid = s * NC + c
    base = wid * ROWS_PER
    pltpu.sync_copy(lab_hbm, lab_v)
    lane = lax.iota(jnp.int32, LANES)
    sems = (sem0, sem1)

    def start(g, b):
        pltpu.async_copy(dist_hbm.at[pl.ds(base + g * RI, RI)],
                         rows_v.at[b], sems[b])

    def wait(g, b):
        pltpu.make_async_copy(dist_hbm.at[pl.ds(base + g * RI, RI)],
                              rows_v.at[b], sems[b]).wait()

    start(0, 0)
    acc = jnp.zeros((LANES,), jnp.float32)
    for g in range(NG):
        b = g % 2
        if g + 1 < NG:
            start(g + 1, 1 - b)
        wait(g, b)
        row0 = h * NROWS + base + g * RI

        def chunk_body(cc, carry):
            ls0 = lab_v[pl.ds(cc * LANES, LANES)]
            ls1 = lab_v[pl.ds(HALF + cc * LANES, LANES)]
            out = []
            for r in range(RI):
                pair = []
                for hh, ls in ((0, ls0), (1, ls1)):
                    bk, bv = carry[r][hh]
                    k = rows_v[b, r, pl.ds(hh * HALF + cc * LANES, LANES)]
                    kd, ld = plsc.sort_key_val(k, ls, descending=True)
                    take_b = bk <= kd
                    mk = jnp.where(take_b, bk, kd)
                    mv = jnp.where(take_b, bv, ld)
                    pair.append(tuple(plsc.sort_key_val(mk, mv)))
                out.append(tuple(pair))
            return tuple(out)

        init = tuple(((jnp.full((LANES,), BIG, jnp.float32),
                       jnp.full((LANES,), -1, jnp.int32)),) * 2
                     for _ in range(RI))
        tops = lax.fori_loop(0, HALF // LANES, chunk_body, init)
        for r in range(RI):
            (fk, fv), (bk2, bv2) = tops[r]
            rk = lax.rev(bk2, (0,))
            rv = lax.rev(bv2, (0,))
            take_f = fk <= rk
            mk = jnp.where(take_f, fk, rk)
            mv = jnp.where(take_f, fv, rv)
            bk, bv = plsc.sort_key_val(mk, mv)
            ridx = jnp.full((LANES,), row0 + r, jnp.int32)
            rowlab = plsc.load_gather(lab_v, [ridx])
            hit = (bv == rowlab) & (lane >= 1) & (lane <= TOPK)
            npop = plsc.all_reduce_population_count(hit)
            acc = acc + jnp.where(npop >= 1, jnp.float32(1.0),
                                  jnp.float32(0.0))

    acc_v[...] = acc
    pltpu.sync_copy(acc_v, out_hbm.at[wid])


def _mean_body(part_ref, out_ref):
    total = jnp.sum(part_ref[...], axis=0, keepdims=True)
    total = jnp.sum(total, axis=1, keepdims=True)
    out_ref[...] = total / jnp.float32(LANES * N)


def _make_sc(h):
    return functools.partial(
        pl.kernel,
        out_type=jax.ShapeDtypeStruct((NW, LANES), jnp.float32),
        mesh=plsc.VectorSubcoreMesh(core_axis_name="c", subcore_axis_name="s"),
        scratch_types=[
            pltpu.VMEM((2, RI, N), jnp.float32),
            pltpu.VMEM((N,), jnp.int32),
            pltpu.VMEM((LANES,), jnp.float32),
            pltpu.SemaphoreType.DMA,
            pltpu.SemaphoreType.DMA,
        ],
        compiler_params=pltpu.CompilerParams(needs_layout_passes=False),
    )(functools.partial(_sc_body, h))


_sc_topk = tuple(_make_sc(h) for h in range(NSPLIT))


def _dist_half(h, embeddings):
    return pl.pallas_call(
        functools.partial(_dist_body, h),
        grid=(NB,),
        in_specs=[
            pl.BlockSpec((BLK, D), lambda i, hh=h: (i + hh * NB, 0)),
            pl.BlockSpec((N, D), lambda i: (0, 0)),
        ],
        out_specs=pl.BlockSpec((BLK, N), lambda i: (i, 0)),
        out_shape=jax.ShapeDtypeStruct((NROWS, N), jnp.float32),
        scratch_shapes=[pltpu.VMEM((1, N), jnp.float32)],
    )(embeddings, embeddings)


@jax.jit
def kernel(embeddings, labels):
    parts = []
    for h in range(NSPLIT):
        dist = _dist_half(h, embeddings)
        parts.append(_sc_topk[h](dist, labels))
    partial_counts = jnp.concatenate(parts, axis=0)
    out = pl.pallas_call(
        _mean_body,
        out_shape=jax.ShapeDtypeStruct((1, 1), jnp.float32),
    )(partial_counts)
    return out.reshape(())


# speedup vs baseline: 1.0506x; 1.0506x over previous
import functools

import jax
import jax.numpy as jnp
from jax import lax
from jax.experimental import pallas as pl
from jax.experimental.pallas import tpu as pltpu
from jax.experimental.pallas import tpu_sc as plsc

N = 4096
D = 128
TOPK = 5
BLK = 256
NSPLIT = 1
NROWS = N // NSPLIT
NB = NROWS // BLK
BIG = 3.0e38

NC = 2
NS = 16
NW = NC * NS
ROWS_PER = NROWS // NW
RI = 8
NG = ROWS_PER // RI
LANES = 16
HALF = N // 2


def _dist_body(h, erow_ref, efull_ref, out_ref, sq_ref):
    i = pl.program_id(0)

    @pl.when(i == 0)
    def _():
        ef = efull_ref[...]
        sq_ref[...] = jnp.sum(ef * ef, axis=1)[None, :]

    er = erow_ref[...]
    sq_rows = jnp.sum(er * er, axis=1)
    dot = jax.lax.dot_general(
        er, efull_ref[...], (((1,), (1,)), ((), ())),
        preferred_element_type=jnp.float32)
    out_ref[...] = (sq_rows[:, None] - 2.0 * dot) + sq_ref[...]


def _sc_body(h, dist_hbm, lab_hbm, out_hbm, rows_v, lab_v, acc_v, sem0, sem1):
    c = lax.axis_index("c")
    s = lax.axis_index("s")
    wid = s * NC + c
    base = wid * ROWS_PER
    pltpu.sync_copy(lab_hbm, lab_v)
    lane = lax.iota(jnp.int32, LANES)
    sems = (sem0, sem1)

    def start(g, b):
        pltpu.async_copy(dist_hbm.at[pl.ds(base + g * RI, RI)],
                         rows_v.at[b], sems[b])

    def wait(g, b):
        pltpu.make_async_copy(dist_hbm.at[pl.ds(base + g * RI, RI)],
                              rows_v.at[b], sems[b]).wait()

    start(0, 0)
    acc = jnp.zeros((LANES,), jnp.float32)
    for g in range(NG):
        b = g % 2
        if g + 1 < NG:
            start(g + 1, 1 - b)
        wait(g, b)
        row0 = h * NROWS + base + g * RI

        def chunk_body(cc, carry):
            ls0 = lab_v[pl.ds(cc * LANES, LANES)]
            ls1 = lab_v[pl.ds(HALF + cc * LANES, LANES)]
            out = []
            for r in range(RI):
                pair = []
                for hh, ls in ((0, ls0), (1, ls1)):
                    bk, bv = carry[r][hh]
                    k = rows_v[b, r, pl.ds(hh * HALF + cc * LANES, LANES)]
                    kd, ld = plsc.sort_key_val(k, ls, descending=True)
                    take_b = bk <= kd
                    mk = jnp.where(take_b, bk, kd)
                    mv = jnp.where(take_b, bv, ld)
                    pair.append(tuple(plsc.sort_key_val(mk, mv)))
                out.append(tuple(pair))
            return tuple(out)

        init = tuple(((jnp.full((LANES,), BIG, jnp.float32),
                       jnp.full((LANES,), -1, jnp.int32)),) * 2
                     for _ in range(RI))
        tops = lax.fori_loop(0, HALF // LANES, chunk_body, init)
        for r in range(RI):
            (fk, fv), (bk2, bv2) = tops[r]
            rk = lax.rev(bk2, (0,))
            rv = lax.rev(bv2, (0,))
            take_f = fk <= rk
            mk = jnp.where(take_f, fk, rk)
            mv = jnp.where(take_f, fv, rv)
            bk, bv = plsc.sort_key_val(mk, mv)
            ridx = jnp.full((LANES,), row0 + r, jnp.int32)
            rowlab = plsc.load_gather(lab_v, [ridx])
            hit = (bv == rowlab) & (lane >= 1) & (lane <= TOPK)
            npop = plsc.all_reduce_population_count(hit)
            acc = acc + jnp.where(npop >= 1, jnp.float32(1.0),
                                  jnp.float32(0.0))

    acc_v[...] = acc
    pltpu.sync_copy(acc_v, out_hbm.at[wid])


def _mean_body(part_ref, out_ref):
    total = jnp.sum(part_ref[...], axis=0, keepdims=True)
    total = jnp.sum(total, axis=1, keepdims=True)
    out_ref[...] = total / jnp.float32(LANES * N)


def _make_sc(h):
    return functools.partial(
        pl.kernel,
        out_type=jax.ShapeDtypeStruct((NW, LANES), jnp.float32),
        mesh=plsc.VectorSubcoreMesh(core_axis_name="c", subcore_axis_name="s"),
        scratch_types=[
            pltpu.VMEM((2, RI, N), jnp.float32),
            pltpu.VMEM((N,), jnp.int32),
            pltpu.VMEM((LANES,), jnp.float32),
            pltpu.SemaphoreType.DMA,
            pltpu.SemaphoreType.DMA,
        ],
        compiler_params=pltpu.CompilerParams(needs_layout_passes=False),
    )(functools.partial(_sc_body, h))


_sc_topk = tuple(_make_sc(h) for h in range(NSPLIT))


def _dist_half(h, embeddings):
    return pl.pallas_call(
        functools.partial(_dist_body, h),
        grid=(NB,),
        in_specs=[
            pl.BlockSpec((BLK, D), lambda i, hh=h: (i + hh * NB, 0)),
            pl.BlockSpec((N, D), lambda i: (0, 0)),
        ],
        out_specs=pl.BlockSpec((BLK, N), lambda i: (i, 0)),
        out_shape=jax.ShapeDtypeStruct((NROWS, N), jnp.float32),
        scratch_shapes=[pltpu.VMEM((1, N), jnp.float32)],
    )(embeddings, embeddings)


@jax.jit
def kernel(embeddings, labels):
    parts = []
    for h in range(NSPLIT):
        dist = _dist_half(h, embeddings)
        parts.append(_sc_topk[h](dist, labels))
    partial_counts = jnp.concatenate(parts, axis=0)
    out = pl.pallas_call(
        _mean_body,
        out_shape=jax.ShapeDtypeStruct((1, 1), jnp.float32),
    )(partial_counts)
    return out.reshape(())


# speedup vs baseline: 1.0734x; 1.0216x over previous
import functools

import jax
import jax.numpy as jnp
from jax import lax
from jax.experimental import pallas as pl
from jax.experimental.pallas import tpu as pltpu
from jax.experimental.pallas import tpu_sc as plsc

N = 4096
D = 128
TOPK = 5
BLK = 512
NSPLIT = 1
NROWS = N // NSPLIT
NB = NROWS // BLK
BIG = 3.0e38

NC = 2
NS = 16
NW = NC * NS
ROWS_PER = NROWS // NW
RI = 8
NG = ROWS_PER // RI
LANES = 16
HALF = N // 2


def _dist_body(h, erow_ref, efull_ref, out_ref, sq_ref):
    i = pl.program_id(0)

    @pl.when(i == 0)
    def _():
        ef = efull_ref[...]
        sq_ref[...] = jnp.sum(ef * ef, axis=1)[None, :]

    er = erow_ref[...]
    sq_rows = jnp.sum(er * er, axis=1)
    dot = jax.lax.dot_general(
        er, efull_ref[...], (((1,), (1,)), ((), ())),
        preferred_element_type=jnp.float32)
    out_ref[...] = (sq_rows[:, None] - 2.0 * dot) + sq_ref[...]


def _sc_body(h, dist_hbm, lab_hbm, out_hbm, rows_v, lab_v, acc_v, sem0, sem1):
    c = lax.axis_index("c")
    s = lax.axis_index("s")
    wid = s * NC + c
    base = wid * ROWS_PER
    pltpu.sync_copy(lab_hbm, lab_v)
    lane = lax.iota(jnp.int32, LANES)
    sems = (sem0, sem1)

    def start(g, b):
        pltpu.async_copy(dist_hbm.at[pl.ds(base + g * RI, RI)],
                         rows_v.at[b], sems[b])

    def wait(g, b):
        pltpu.make_async_copy(dist_hbm.at[pl.ds(base + g * RI, RI)],
                              rows_v.at[b], sems[b]).wait()

    start(0, 0)
    acc = jnp.zeros((LANES,), jnp.float32)
    for g in range(NG):
        b = g % 2
        if g + 1 < NG:
            start(g + 1, 1 - b)
        wait(g, b)
        row0 = h * NROWS + base + g * RI

        def chunk_body(cc, carry):
            ls0 = lab_v[pl.ds(cc * LANES, LANES)]
            ls1 = lab_v[pl.ds(HALF + cc * LANES, LANES)]
            out = []
            for r in range(RI):
                pair = []
                for hh, ls in ((0, ls0), (1, ls1)):
                    bk, bv = carry[r][hh]
                    k = rows_v[b, r, pl.ds(hh * HALF + cc * LANES, LANES)]
                    kd, ld = plsc.sort_key_val(k, ls, descending=True)
                    take_b = bk <= kd
                    mk = jnp.where(take_b, bk, kd)
                    mv = jnp.where(take_b, bv, ld)
                    pair.append(tuple(plsc.sort_key_val(mk, mv)))
                out.append(tuple(pair))
            return tuple(out)

        init = tuple(((jnp.full((LANES,), BIG, jnp.float32),
                       jnp.full((LANES,), -1, jnp.int32)),) * 2
                     for _ in range(RI))
        tops = lax.fori_loop(0, HALF // LANES, chunk_body, init)
        for r in range(RI):
            (fk, fv), (bk2, bv2) = tops[r]
            rk = lax.rev(bk2, (0,))
            rv = lax.rev(bv2, (0,))
            take_f = fk <= rk
            mk = jnp.where(take_f, fk, rk)
            mv = jnp.where(take_f, fv, rv)
            bk, bv = plsc.sort_key_val(mk, mv)
            ridx = jnp.full((LANES,), row0 + r, jnp.int32)
            rowlab = plsc.load_gather(lab_v, [ridx])
            hit = (bv == rowlab) & (lane >= 1) & (lane <= TOPK)
            npop = plsc.all_reduce_population_count(hit)
            acc = acc + jnp.where(npop >= 1, jnp.float32(1.0),
                                  jnp.float32(0.0))

    acc_v[...] = acc
    pltpu.sync_copy(acc_v, out_hbm.at[wid])


def _mean_body(part_ref, out_ref):
    total = jnp.sum(part_ref[...], axis=0, keepdims=True)
    total = jnp.sum(total, axis=1, keepdims=True)
    out_ref[...] = total / jnp.float32(LANES * N)


def _make_sc(h):
    return functools.partial(
        pl.kernel,
        out_type=jax.ShapeDtypeStruct((NW, LANES), jnp.float32),
        mesh=plsc.VectorSubcoreMesh(core_axis_name="c", subcore_axis_name="s"),
        scratch_types=[
            pltpu.VMEM((2, RI, N), jnp.float32),
            pltpu.VMEM((N,), jnp.int32),
            pltpu.VMEM((LANES,), jnp.float32),
            pltpu.SemaphoreType.DMA,
            pltpu.SemaphoreType.DMA,
        ],
        compiler_params=pltpu.CompilerParams(needs_layout_passes=False),
    )(functools.partial(_sc_body, h))


_sc_topk = tuple(_make_sc(h) for h in range(NSPLIT))


def _dist_half(h, embeddings):
    return pl.pallas_call(
        functools.partial(_dist_body, h),
        grid=(NB,),
        in_specs=[
            pl.BlockSpec((BLK, D), lambda i, hh=h: (i + hh * NB, 0)),
            pl.BlockSpec((N, D), lambda i: (0, 0)),
        ],
        out_specs=pl.BlockSpec((BLK, N), lambda i: (i, 0)),
        out_shape=jax.ShapeDtypeStruct((NROWS, N), jnp.float32),
        scratch_shapes=[pltpu.VMEM((1, N), jnp.float32)],
    )(embeddings, embeddings)


@jax.jit
def kernel(embeddings, labels):
    parts = []
    for h in range(NSPLIT):
        dist = _dist_half(h, embeddings)
        parts.append(_sc_topk[h](dist, labels))
    partial_counts = jnp.concatenate(parts, axis=0)
    out = pl.pallas_call(
        _mean_body,
        out_shape=jax.ShapeDtypeStruct((1, 1), jnp.float32),
    )(partial_counts)
    return out.reshape(())


# speedup vs baseline: 1.0759x; 1.0024x over previous
import functools

import jax
import jax.numpy as jnp
from jax import lax
from jax.experimental import pallas as pl
from jax.experimental.pallas import tpu as pltpu
from jax.experimental.pallas import tpu_sc as plsc

N = 4096
D = 128
TOPK = 5
BLK = 512
NSPLIT = 1
NROWS = N // NSPLIT
NB = NROWS // BLK
BIG = 3.0e38

NC = 2
NS = 16
NW = NC * NS
ROWS_PER = NROWS // NW
RI = 8
NG = ROWS_PER // RI
LANES = 16
HALF = N // 2


def _dist_body(h, erow_ref, efull_ref, out_ref, sq_ref):
    i = pl.program_id(0)

    @pl.when(i == 0)
    def _():
        ef = efull_ref[...]
        sq_ref[...] = jnp.sum(ef * ef, axis=1)[None, :]

    er = erow_ref[...]
    sq_rows = jnp.sum(er * er, axis=1)
    dot = jax.lax.dot_general(
        er, efull_ref[...], (((1,), (1,)), ((), ())),
        preferred_element_type=jnp.float32)
    out_ref[...] = (sq_rows[:, None] - 2.0 * dot) + sq_ref[...]


def _sc_body(h, dist_hbm, lab_hbm, out_hbm, rows_v, lab_v, acc_v, sem0, sem1):
    c = lax.axis_index("c")
    s = lax.axis_index("s")
    wid = s * NC + c
    base = wid * ROWS_PER
    pltpu.sync_copy(lab_hbm, lab_v)
    lane = lax.iota(jnp.int32, LANES)
    sems = (sem0, sem1)

    def start(g, b):
        pltpu.async_copy(dist_hbm.at[pl.ds(base + g * RI, RI)],
                         rows_v.at[b], sems[b])

    def wait(g, b):
        pltpu.make_async_copy(dist_hbm.at[pl.ds(base + g * RI, RI)],
                              rows_v.at[b], sems[b]).wait()

    start(0, 0)
    acc = jnp.zeros((LANES,), jnp.float32)
    for g in range(NG):
        b = g % 2
        if g + 1 < NG:
            start(g + 1, 1 - b)
        wait(g, b)
        row0 = h * NROWS + base + g * RI

        def chunk_body(cc, carry):
            ls0 = lab_v[pl.ds(cc * LANES, LANES)]
            ls1 = lab_v[pl.ds(HALF + cc * LANES, LANES)]
            out = []
            for r in range(RI):
                pair = []
                for hh, ls in ((0, ls0), (1, ls1)):
                    bk, bv = carry[r][hh]
                    k = rows_v[b, r, pl.ds(hh * HALF + cc * LANES, LANES)]
                    kd, ld = plsc.sort_key_val(k, ls, descending=True)
                    take_b = bk <= kd
                    mk = jnp.where(take_b, bk, kd)
                    mv = jnp.where(take_b, bv, ld)
                    pair.append(tuple(plsc.sort_key_val(mk, mv)))
                out.append(tuple(pair))
            return tuple(out)

        init = tuple(((jnp.full((LANES,), BIG, jnp.float32),
                       jnp.full((LANES,), -1, jnp.int32)),) * 2
                     for _ in range(RI))
        tops = plsc.parallel_loop(0, HALF // LANES, carry=init)(chunk_body)
        for r in range(RI):
            (fk, fv), (bk2, bv2) = tops[r]
            rk = lax.rev(bk2, (0,))
            rv = lax.rev(bv2, (0,))
            take_f = fk <= rk
            mk = jnp.where(take_f, fk, rk)
            mv = jnp.where(take_f, fv, rv)
            bk, bv = plsc.sort_key_val(mk, mv)
            ridx = jnp.full((LANES,), row0 + r, jnp.int32)
            rowlab = plsc.load_gather(lab_v, [ridx])
            hit = (bv == rowlab) & (lane >= 1) & (lane <= TOPK)
            npop = plsc.all_reduce_population_count(hit)
            acc = acc + jnp.where(npop >= 1, jnp.float32(1.0),
                                  jnp.float32(0.0))

    acc_v[...] = acc
    pltpu.sync_copy(acc_v, out_hbm.at[wid])


def _mean_body(part_ref, out_ref):
    total = jnp.sum(part_ref[...], axis=0, keepdims=True)
    total = jnp.sum(total, axis=1, keepdims=True)
    out_ref[...] = total / jnp.float32(LANES * N)


def _make_sc(h):
    return functools.partial(
        pl.kernel,
        out_type=jax.ShapeDtypeStruct((NW, LANES), jnp.float32),
        mesh=plsc.VectorSubcoreMesh(core_axis_name="c", subcore_axis_name="s"),
        scratch_types=[
            pltpu.VMEM((2, RI, N), jnp.float32),
            pltpu.VMEM((N,), jnp.int32),
            pltpu.VMEM((LANES,), jnp.float32),
            pltpu.SemaphoreType.DMA,
            pltpu.SemaphoreType.DMA,
        ],
        compiler_params=pltpu.CompilerParams(needs_layout_passes=False),
    )(functools.partial(_sc_body, h))


_sc_topk = tuple(_make_sc(h) for h in range(NSPLIT))


def _dist_half(h, embeddings):
    return pl.pallas_call(
        functools.partial(_dist_body, h),
        grid=(NB,),
        in_specs=[
            pl.BlockSpec((BLK, D), lambda i, hh=h: (i + hh * NB, 0)),
            pl.BlockSpec((N, D), lambda i: (0, 0)),
        ],
        out_specs=pl.BlockSpec((BLK, N), lambda i: (i, 0)),
        out_shape=jax.ShapeDtypeStruct((NROWS, N), jnp.float32),
        scratch_shapes=[pltpu.VMEM((1, N), jnp.float32)],
    )(embeddings, embeddings)


@jax.jit
def kernel(embeddings, labels):
    parts = []
    for h in range(NSPLIT):
        dist = _dist_half(h, embeddings)
        parts.append(_sc_topk[h](dist, labels))
    partial_counts = jnp.concatenate(parts, axis=0)
    out = pl.pallas_call(
        _mean_body,
        out_shape=jax.ShapeDtypeStruct((1, 1), jnp.float32),
    )(partial_counts)
    return out.reshape(())
